# baseline (speedup 1.0000x reference)
import functools

import jax
import jax.numpy as jnp
from jax import lax
from jax.experimental import pallas as pl
from jax.experimental.pallas import tpu as pltpu
from jax.experimental.pallas import tpu_sc as plsc

VOCAB = 100000
DIM = 128
BLOCK = 4096
N_BLOCKS = (VOCAB * DIM) // BLOCK
ROWS_PER_ABSMAX = BLOCK // DIM
BATCH = 4096
FIELDS = 26

NC, NS, L = 2, 16, 16
NW = NC * NS


def _build(n_rows, chunk, vocab, n_blocks, depth):
    bpw = n_rows // NW
    nchunk = bpw // chunk
    mesh = plsc.VectorSubcoreMesh(core_axis_name="c", subcore_axis_name="s")

    @functools.partial(
        pl.kernel,
        out_type=jax.ShapeDtypeStruct((n_rows, DIM), jnp.float32),
        mesh=mesh,
        compiler_params=pltpu.CompilerParams(
            use_tc_tiling_on_sc=False, needs_layout_passes=False),
        scratch_types=[
            pltpu.VMEM((nchunk, chunk), jnp.int32),
            pltpu.VMEM((n_blocks,), jnp.float32),
            pltpu.VMEM((256,), jnp.float32),
            pltpu.VMEM((depth, chunk, DIM), jnp.int32),
            pltpu.VMEM((depth, chunk, DIM), jnp.float32),
        ] + [pltpu.SemaphoreType.DMA] * (2 * depth),
    )
    def k(x_hbm, qw_hbm, absmax_hbm, code_hbm, out_hbm,
          idx_v, absmax_v, code_v, rows_v, out_v, *sems):
        in_sems = list(sems[:depth])
        out_sems = list(sems[depth:])
        wid = lax.axis_index("s") * NC + lax.axis_index("c")
        pltpu.sync_copy(x_hbm.at[wid], idx_v)
        pltpu.sync_copy(absmax_hbm, absmax_v)
        pltpu.sync_copy(code_hbm, code_v)

        for b in range(depth):
            pltpu.async_copy(qw_hbm.at[idx_v.at[b]], rows_v.at[b], in_sems[b])

        def dequant_chunk(c, b):
            @plsc.parallel_loop(0, chunk // L)
            def group_body(g):
                idx16 = idx_v[c, pl.ds(g * L, L)]
                shift5 = jnp.full((L,), 5, jnp.int32)
                s_g = plsc.load_gather(
                    absmax_v, [lax.shift_right_logical(idx16, shift5)])
                for j in range(L):
                    scale = jnp.take(s_g, jnp.full((L,), j, jnp.int32))
                    r = g * L + j
                    for kk in range(DIM // L):
                        q = rows_v[b, r, pl.ds(kk * L, L)]
                        out_v[b, r, pl.ds(kk * L, L)] = (
                            plsc.load_gather(code_v, [q]) * scale)

        def ring_body(g, carry):
            for b in range(depth):
                c = g * depth + b
                pltpu.make_async_copy(
                    qw_hbm.at[pl.ds(0, chunk)], rows_v.at[b],
                    in_sems[b]).wait()
                @pl.when(c >= depth)
                def _():
                    pltpu.make_async_copy(
                        out_v.at[b], out_hbm.at[pl.ds(0, chunk)],
                        out_sems[b]).wait()

                dequant_chunk(c, b)

                pltpu.async_copy(
                    out_v.at[b],
                    out_hbm.at[pl.ds(wid * bpw + c * chunk, chunk)],
                    out_sems[b])
                @pl.when(c + depth < nchunk)
                def _():
                    pltpu.async_copy(
                        qw_hbm.at[idx_v.at[c + depth]], rows_v.at[b],
                        in_sems[b])
            return carry

        lax.fori_loop(0, nchunk // depth, ring_body, 0, unroll=False)

        for b in range(depth):
            pltpu.make_async_copy(
                out_v.at[b], out_hbm.at[pl.ds(0, chunk)], out_sems[b]).wait()

    return k


_CHUNK = 64
_KERNEL = _build(BATCH * FIELDS, _CHUNK, VOCAB, N_BLOCKS, 4)


def kernel(x, qweight, absmax, code):
    n_rows = BATCH * FIELDS
    bpw = n_rows // NW
    xr = x.reshape(NW, bpw // _CHUNK, _CHUNK)
    out = _KERNEL(xr, qweight, absmax, code)
    return out.reshape(BATCH, FIELDS, DIM)


# speedup vs baseline: 1.1894x; 1.1894x over previous
import functools

import jax
import jax.numpy as jnp
from jax import lax
from jax.experimental import pallas as pl
from jax.experimental.pallas import tpu as pltpu
from jax.experimental.pallas import tpu_sc as plsc

VOCAB = 100000
DIM = 128
BLOCK = 4096
N_BLOCKS = (VOCAB * DIM) // BLOCK
ROWS_PER_ABSMAX = BLOCK // DIM
BATCH = 4096
FIELDS = 26

NC, NS, L = 2, 16, 16
NW = NC * NS


def _build(n_rows, chunk, vocab, n_blocks, depth):
    bpw = n_rows // NW
    nchunk = bpw // chunk
    assert nchunk % depth == 0 and chunk % L == 0
    mesh = plsc.VectorSubcoreMesh(core_axis_name="c", subcore_axis_name="s")

    @functools.partial(
        pl.kernel,
        out_type=jax.ShapeDtypeStruct((n_rows, DIM), jnp.float32),
        mesh=mesh,
        compiler_params=pltpu.CompilerParams(
            use_tc_tiling_on_sc=False, needs_layout_passes=False),
        scratch_types=[
            pltpu.VMEM((nchunk, chunk), jnp.int32),
            pltpu.VMEM((n_blocks,), jnp.float32),
            pltpu.VMEM((256,), jnp.float32),
            pltpu.VMEM((depth, chunk, DIM), jnp.float32),
        ] + [pltpu.SemaphoreType.DMA] * (2 * depth),
    )
    def k(x_hbm, qw_hbm, absmax_hbm, code_hbm, out_hbm,
          idx_v, absmax_v, code_v, rows_v, *sems):
        in_sems = list(sems[:depth])
        out_sems = list(sems[depth:])
        wid = lax.axis_index("s") * NC + lax.axis_index("c")
        pltpu.sync_copy(x_hbm.at[wid], idx_v)
        pltpu.sync_copy(absmax_hbm, absmax_v)
        pltpu.sync_copy(code_hbm, code_v)

        for b in range(2):
            pltpu.async_copy(qw_hbm.at[idx_v.at[b]], rows_v.at[b], in_sems[b])

        def dequant_chunk(c, b):
            @plsc.parallel_loop(0, chunk // L)
            def group_body(g):
                idx16 = idx_v[c, pl.ds(g * L, L)]
                shift5 = jnp.full((L,), 5, jnp.int32)
                s_g = plsc.load_gather(
                    absmax_v, [lax.shift_right_logical(idx16, shift5)])
                for j in range(L):
                    scale = jnp.take(s_g, jnp.full((L,), j, jnp.int32))
                    r = g * L + j
                    for kk in range(DIM // L):
                        q = lax.bitcast_convert_type(
                            rows_v[b, r, pl.ds(kk * L, L)], jnp.int32)
                        rows_v[b, r, pl.ds(kk * L, L)] = (
                            plsc.load_gather(code_v, [q]) * scale)

        def ring_body(g, carry):
            for b in range(depth):
                c = g * depth + b
                bn = (b + 2) % depth
                @pl.when(jnp.logical_and(c + 2 >= depth, c + 2 < nchunk))
                def _():
                    pltpu.make_async_copy(
                        rows_v.at[bn], out_hbm.at[pl.ds(0, chunk)],
                        out_sems[bn]).wait()
                @pl.when(c + 2 < nchunk)
                def _():
                    pltpu.async_copy(
                        qw_hbm.at[idx_v.at[c + 2]], rows_v.at[bn],
                        in_sems[bn])
                pltpu.make_async_copy(
                    qw_hbm.at[pl.ds(0, chunk)], rows_v.at[b],
                    in_sems[b]).wait()

                dequant_chunk(c, b)

                pltpu.async_copy(
                    rows_v.at[b],
                    out_hbm.at[pl.ds(wid * bpw + c * chunk, chunk)],
                    out_sems[b])
            return carry

        lax.fori_loop(0, nchunk // depth, ring_body, 0, unroll=False)

        for b in range(depth):
            pltpu.make_async_copy(
                rows_v.at[b], out_hbm.at[pl.ds(0, chunk)], out_sems[b]).wait()

    return k


_CHUNK = 208
_KERNEL = _build(BATCH * FIELDS, _CHUNK, VOCAB, N_BLOCKS, 4)


def kernel(x, qweight, absmax, code):
    n_rows = BATCH * FIELDS
    bpw = n_rows // NW
    xr = x.reshape(NW, bpw // _CHUNK, _CHUNK)
    qw_f = lax.bitcast_convert_type(qweight, jnp.float32)
    out = _KERNEL(xr, qw_f, absmax, code)
    return out.reshape(BATCH, FIELDS, DIM)


# speedup vs baseline: 1.4255x; 1.1985x over previous
import functools

import jax
import jax.numpy as jnp
from jax import lax
from jax.experimental import pallas as pl
from jax.experimental.pallas import tpu as pltpu
from jax.experimental.pallas import tpu_sc as plsc

VOCAB = 100000
DIM = 128
BLOCK = 4096
N_BLOCKS = (VOCAB * DIM) // BLOCK
ROWS_PER_ABSMAX = BLOCK // DIM
BATCH = 4096
FIELDS = 26

NC, NS, L = 2, 16, 16
NW = NC * NS


def _build(n_rows, chunk, vocab, n_blocks, depth):
    bpw = n_rows // NW
    nchunk = bpw // chunk
    assert nchunk % depth == 0 and chunk % L == 0
    mesh = plsc.VectorSubcoreMesh(core_axis_name="c", subcore_axis_name="s")

    @functools.partial(
        pl.kernel,
        out_type=jax.ShapeDtypeStruct((n_rows, DIM), jnp.float32),
        mesh=mesh,
        compiler_params=pltpu.CompilerParams(
            use_tc_tiling_on_sc=False, needs_layout_passes=False),
        scratch_types=[
            pltpu.VMEM((nchunk, chunk), jnp.int32),
            pltpu.VMEM((n_blocks,), jnp.float32),
            pltpu.VMEM((256,), jnp.float32),
            pltpu.VMEM((depth, chunk, DIM), jnp.int32),
            pltpu.VMEM((depth, chunk, DIM), jnp.float32),
        ] + [pltpu.SemaphoreType.DMA] * (2 * depth),
    )
    def k(x_hbm, qw_hbm, absmax_hbm, code_hbm, out_hbm,
          idx_v, absmax_v, code_v, rows_v, out_v, *sems):
        in_sems = list(sems[:depth])
        out_sems = list(sems[depth:])
        wid = lax.axis_index("s") * NC + lax.axis_index("c")
        pltpu.sync_copy(x_hbm.at[wid], idx_v)
        pltpu.sync_copy(absmax_hbm, absmax_v)
        pltpu.sync_copy(code_hbm, code_v)

        for b in range(depth):
            pltpu.async_copy(qw_hbm.at[idx_v.at[b]], rows_v.at[b], in_sems[b])

        def dequant_chunk(c, b):
            @plsc.parallel_loop(0, chunk // L)
            def group_body(g):
                idx16 = idx_v[c, pl.ds(g * L, L)]
                shift5 = jnp.full((L,), 5, jnp.int32)
                s_g = plsc.load_gather(
                    absmax_v, [lax.shift_right_logical(idx16, shift5)])
                for j in range(L):
                    scale = jnp.take(s_g, jnp.full((L,), j, jnp.int32))
                    r = g * L + j
                    for kk in range(DIM // L):
                        q = rows_v[b, r, pl.ds(kk * L, L)]
                        out_v[b, r, pl.ds(kk * L, L)] = (
                            plsc.load_gather(code_v, [q]) * scale)

        def ring_body(g, carry):
            for b in range(depth):
                c = g * depth + b
                pltpu.make_async_copy(
                    qw_hbm.at[pl.ds(0, chunk)], rows_v.at[b],
                    in_sems[b]).wait()
                @pl.when(c >= depth)
                def _():
                    pltpu.make_async_copy(
                        out_v.at[b], out_hbm.at[pl.ds(0, chunk)],
                        out_sems[b]).wait()

                dequant_chunk(c, b)

                pltpu.async_copy(
                    out_v.at[b],
                    out_hbm.at[pl.ds(wid * bpw + c * chunk, chunk)],
                    out_sems[b])
                @pl.when(c + depth < nchunk)
                def _():
                    pltpu.async_copy(
                        qw_hbm.at[idx_v.at[c + depth]], rows_v.at[b],
                        in_sems[b])
            return carry

        lax.fori_loop(0, nchunk // depth, ring_body, 0, unroll=False)

        for b in range(depth):
            pltpu.make_async_copy(
                out_v.at[b], out_hbm.at[pl.ds(0, chunk)], out_sems[b]).wait()

    return k


_CHUNK = 208
_KERNEL = _build(BATCH * FIELDS, _CHUNK, VOCAB, N_BLOCKS, 2)


def kernel(x, qweight, absmax, code):
    n_rows = BATCH * FIELDS
    bpw = n_rows // NW
    xr = x.reshape(NW, bpw // _CHUNK, _CHUNK)
    out = _KERNEL(xr, qweight, absmax, code)
    return out.reshape(BATCH, FIELDS, DIM)
